# baseline (speedup 1.0000x reference)
import functools

import jax
import jax.numpy as jnp
from jax import lax
from jax.experimental import pallas as pl
from jax.experimental.pallas import tpu as pltpu
from jax.experimental.pallas import tpu_sc as plsc

_MASK_RATIO = 0.15
_LANES = 16
_NWORKERS = 32
_RB = 4


def _sc_mul_kernel(t, rows_per_w, idx_pad, x_hbm, idx_hbm, out_hbm,
                   bin0, bin1, bout0, bout1, mask_v, idx_v,
                   sin0, sin1, sout0, sout1):
    wid = lax.axis_index("s") * 2 + lax.axis_index("c")
    row0 = wid * rows_per_w

    pltpu.sync_copy(idx_hbm, idx_v)
    ones = jnp.ones((_LANES,), jnp.float32)
    for i in range((t + _LANES) // _LANES):
        mask_v[pl.ds(i * _LANES, _LANES)] = ones
    zeros = jnp.zeros((_LANES,), jnp.float32)
    for i in range(idx_pad // _LANES):
        iv = idx_v[pl.ds(i * _LANES, _LANES)]
        plsc.store_scatter(mask_v, [iv], zeros)

    bins, bouts = [bin0, bin1], [bout0, bout1]
    sins, souts = [sin0, sin1], [sout0, sout1]
    nch = rows_per_w // _RB

    def in_copy(g):
        return pltpu.async_copy(
            x_hbm.at[pl.ds(row0 + g * _RB, _RB)], bins[g % 2], sins[g % 2])

    def out_copy(g):
        return pltpu.async_copy(
            bouts[g % 2], out_hbm.at[pl.ds(row0 + g * _RB, _RB)], souts[g % 2])

    pending_in = {0: in_copy(0), 1: in_copy(1)}
    pending_out = {}
    for g in range(nch):
        pending_in.pop(g).wait()
        if g >= 2:
            pending_out.pop(g - 2).wait()
        bi, bo = bins[g % 2], bouts[g % 2]

        def body(j, _, bi=bi, bo=bo):
            off = j * _LANES
            mv = mask_v[pl.ds(off, _LANES)]
            for r in range(_RB):
                bo[r, pl.ds(off, _LANES)] = bi[r, pl.ds(off, _LANES)] * mv
            return 0

        lax.fori_loop(0, t // _LANES, body, 0)
        pending_out[g] = out_copy(g)
        if g + 2 < nch:
            pending_in[g + 2] = in_copy(g + 2)
    for g in sorted(pending_out):
        pending_out.pop(g).wait()


def kernel(x):
    B, C, T = x.shape
    n_mask = int(T * _MASK_RATIO)
    if n_mask <= 0:
        return x

    import numpy as np
    with jax.ensure_compile_time_eval():
        key = jax.random.fold_in(jax.random.key(0), 1)
        mask_indices = np.asarray(
            jax.random.permutation(key, T)[:n_mask], dtype=np.int32
        )
    idx_pad = ((n_mask + _LANES - 1) // _LANES) * _LANES
    idx_np = np.full((idx_pad,), T, dtype=np.int32)
    idx_np[:n_mask] = mask_indices
    idx1d = jnp.asarray(idx_np)

    rows = B * C
    rows_per_w = rows // _NWORKERS
    xr = x.reshape(rows, T)

    mesh = plsc.VectorSubcoreMesh(core_axis_name="c", subcore_axis_name="s")
    sc_mul = functools.partial(
        pl.kernel,
        mesh=mesh,
        out_type=jax.ShapeDtypeStruct((rows, T), jnp.float32),
        scratch_types=[
            pltpu.VMEM((_RB, T), jnp.float32),
            pltpu.VMEM((_RB, T), jnp.float32),
            pltpu.VMEM((_RB, T), jnp.float32),
            pltpu.VMEM((_RB, T), jnp.float32),
            pltpu.VMEM((T + _LANES,), jnp.float32),
            pltpu.VMEM((idx_pad,), jnp.int32),
            pltpu.SemaphoreType.DMA,
            pltpu.SemaphoreType.DMA,
            pltpu.SemaphoreType.DMA,
            pltpu.SemaphoreType.DMA,
        ],
        compiler_params=pltpu.CompilerParams(needs_layout_passes=False),
    )(functools.partial(_sc_mul_kernel, T, rows_per_w, idx_pad))

    out = sc_mul(xr, idx1d)
    return out.reshape(B, C, T)


# speedup vs baseline: 1.9255x; 1.9255x over previous
import jax
import jax.numpy as jnp
import numpy as np
from jax import lax
from jax.experimental import pallas as pl
from jax.experimental.pallas import tpu as pltpu

_MASK_RATIO = 0.15
_ROW_BLOCK = 512
_T = 4096
_N_MASK = int(_T * _MASK_RATIO)
_IDX_PAD = ((_N_MASK + 7) // 8) * 8

_IDX2D_CACHE = None


def _mask_idx2d() -> np.ndarray:
    global _IDX2D_CACHE
    if _IDX2D_CACHE is None:
        with jax.ensure_compile_time_eval():
            key = jax.random.fold_in(jax.random.key(0), 1)
            idx = np.asarray(
                jax.random.permutation(key, _T)[:_N_MASK], dtype=np.int32
            )
        arr = np.full((_IDX_PAD, 1), _T, dtype=np.int32)
        arr[:_N_MASK, 0] = idx
        _IDX2D_CACHE = arr
    return _IDX2D_CACHE


def _mask_mul_kernel(idx_ref, x_ref, o_ref, mask_ref):
    @pl.when(pl.program_id(0) == 0)
    def _build():
        idx = idx_ref[...]
        t_iota = lax.broadcasted_iota(jnp.int32, (idx.shape[0], mask_ref.shape[1]), 1)
        hit = jnp.any(idx == t_iota, axis=0, keepdims=True)
        mask_ref[...] = jnp.where(hit, 0.0, 1.0).astype(jnp.float32)

    o_ref[...] = x_ref[...] * mask_ref[...]


def kernel(x):
    B, C, T = x.shape
    n_mask = int(T * _MASK_RATIO)
    if n_mask <= 0:
        return x
    assert T == _T

    rows = B * C
    xr = x.reshape(rows, T)
    grid = (rows // _ROW_BLOCK,)

    out = pl.pallas_call(
        _mask_mul_kernel,
        grid=grid,
        in_specs=[
            pl.BlockSpec((_IDX_PAD, 1), lambda i: (0, 0)),
            pl.BlockSpec((_ROW_BLOCK, T), lambda i: (i, 0)),
        ],
        out_specs=pl.BlockSpec((_ROW_BLOCK, T), lambda i: (i, 0)),
        out_shape=jax.ShapeDtypeStruct((rows, T), x.dtype),
        scratch_shapes=[pltpu.VMEM((1, T), jnp.float32)],
    )(jnp.asarray(_mask_idx2d()), xr)
    return out.reshape(B, C, T)
